# baseline (speedup 1.0000x reference)
import functools

import jax
import jax.numpy as jnp
from jax import lax
from jax.experimental import pallas as pl
from jax.experimental.pallas import tpu as pltpu
from jax.experimental.pallas import tpu_sc as plsc

D_MODEL = 128
CHUNK = 128
NBUF = 6
LOOKAHEAD = 4


def _gather_sc(table, idx3):
    NW, n_ch, CH = idx3.shape
    b_per_w = n_ch * CH
    B = NW * b_per_w
    info = plsc.get_sparse_core_info()
    NC = info.num_cores
    mesh = plsc.VectorSubcoreMesh(core_axis_name="c", subcore_axis_name="s")
    K = LOOKAHEAD
    assert K <= NBUF - 1 and n_ch > 2 * K and (n_ch - 2 * K) % NBUF == 0

    @functools.partial(
        pl.kernel,
        mesh=mesh,
        out_type=jax.ShapeDtypeStruct((B, D_MODEL), jnp.float32),
        scratch_types=[
            pltpu.VMEM((n_ch, CH), jnp.int32),
            pltpu.VMEM((NBUF, CH, D_MODEL), jnp.float32),
            pltpu.VMEM_SHARED(table.shape, jnp.float32),
            pltpu.SemaphoreType.DMA,
        ]
        + [pltpu.SemaphoreType.DMA] * (2 * NBUF),
    )
    def k(table_hbm, idx_hbm, out_hbm, idx_v, rows_v, tab_sh, tsem, *sems):
        gsems, osems = sems[:NBUF], sems[NBUF:]
        sid = lax.axis_index("s")
        wid = sid * NC + lax.axis_index("c")

        @pl.when(sid == 0)
        def _():
            pltpu.async_copy(table_hbm, tab_sh, tsem)

        pltpu.sync_copy(idx_hbm.at[wid], idx_v)

        @pl.when(sid == 0)
        def _():
            pltpu.make_async_copy(table_hbm, tab_sh, tsem).wait()

        plsc.subcore_barrier()

        base = wid * b_per_w

        def _src(b):
            return table_hbm if b == 0 else tab_sh

        def start_gather(j, b):
            pltpu.async_copy(_src(b).at[idx_v.at[j]], rows_v.at[b], gsems[b])

        def wait_gather(j, b):
            pltpu.make_async_copy(
                _src(b).at[idx_v.at[j]], rows_v.at[b], gsems[b]
            ).wait()

        def start_out(j, b):
            pltpu.async_copy(
                rows_v.at[b], out_hbm.at[pl.ds(base + j * CH, CH)], osems[b]
            )

        def wait_out(j, b):
            pltpu.make_async_copy(
                rows_v.at[b], out_hbm.at[pl.ds(base + j * CH, CH)], osems[b]
            ).wait()


        for j in range(K):
            start_gather(j, j % NBUF)
        for j in range(K):
            if j + K - NBUF >= 0:
                wait_out(j + K - NBUF, (j + K) % NBUF)
            start_gather(j + K, (j + K) % NBUF)
            wait_gather(j, j % NBUF)
            start_out(j, j % NBUF)

        def body(gg, carry):
            for p in range(NBUF):
                j = NBUF * gg + K + p
                bg = (K + p + K) % NBUF
                bj = (K + p) % NBUF
                wait_out(j + K - NBUF, bg)
                start_gather(j + K, bg)
                wait_gather(j, bj)
                start_out(j, bj)
            return carry

        lax.fori_loop(0, (n_ch - 2 * K) // NBUF, body, 0)

        for j in range(n_ch - K, n_ch):
            wait_out(j + K - NBUF, (j + K) % NBUF)
            wait_gather(j, j % NBUF)
            start_out(j, j % NBUF)
        for j in range(n_ch - NBUF + K, n_ch):
            wait_out(j, j % NBUF)

    return k(table, idx3)


def kernel(pos_seq, table):
    B4, S = pos_seq.shape
    B = B4 * S
    NW = 32
    idx3 = pos_seq.astype(jnp.int32).reshape(NW, (B // NW) // CHUNK, CHUNK)
    out = _gather_sc(table, idx3)
    return out.reshape(B4, S, D_MODEL)


# speedup vs baseline: 1.7716x; 1.7716x over previous
import functools

import jax
import jax.numpy as jnp
from jax import lax
from jax.experimental import pallas as pl
from jax.experimental.pallas import tpu as pltpu
from jax.experimental.pallas import tpu_sc as plsc

D_MODEL = 128
CHUNK = 128
NBUF = 6
LOOKAHEAD = 4


def _gather_sc(table, idx3):
    NW, n_ch, CH = idx3.shape
    b_per_w = n_ch * CH
    B = NW * b_per_w
    info = plsc.get_sparse_core_info()
    NC = info.num_cores
    mesh = plsc.VectorSubcoreMesh(core_axis_name="c", subcore_axis_name="s")
    K = LOOKAHEAD
    assert K <= NBUF - 1 and n_ch > 2 * K and (n_ch - 2 * K) % NBUF == 0

    @functools.partial(
        pl.kernel,
        mesh=mesh,
        out_type=jax.ShapeDtypeStruct((B, D_MODEL), jnp.float32),
        scratch_types=[
            pltpu.VMEM((n_ch, CH), jnp.int32),
            pltpu.VMEM((NBUF, CH, D_MODEL), jnp.float32),
            pltpu.VMEM_SHARED(table.shape, jnp.float32),
            pltpu.SemaphoreType.DMA,
        ]
        + [pltpu.SemaphoreType.DMA] * (2 * NBUF),
    )
    def k(table_hbm, idx_hbm, out_hbm, idx_v, rows_v, tab_sh, tsem, *sems):
        gsems, osems = sems[:NBUF], sems[NBUF:]
        sid = lax.axis_index("s")
        wid = sid * NC + lax.axis_index("c")

        @pl.when(sid == 0)
        def _():
            pltpu.async_copy(table_hbm, tab_sh, tsem)

        pltpu.sync_copy(idx_hbm.at[wid], idx_v)

        @pl.when(sid == 0)
        def _():
            pltpu.make_async_copy(table_hbm, tab_sh, tsem).wait()

        plsc.subcore_barrier()

        base = wid * b_per_w

        def start_gather(j, b):
            pltpu.async_copy(tab_sh.at[idx_v.at[j]], rows_v.at[b], gsems[b])

        def wait_gather(j, b):
            pltpu.make_async_copy(
                tab_sh.at[idx_v.at[j]], rows_v.at[b], gsems[b]
            ).wait()

        def start_out(j, b):
            pltpu.async_copy(
                rows_v.at[b], out_hbm.at[pl.ds(base + j * CH, CH)], osems[b]
            )

        def wait_out(j, b):
            pltpu.make_async_copy(
                rows_v.at[b], out_hbm.at[pl.ds(base + j * CH, CH)], osems[b]
            ).wait()


        for j in range(K):
            start_gather(j, j % NBUF)
        for j in range(K):
            if j + K - NBUF >= 0:
                wait_out(j + K - NBUF, (j + K) % NBUF)
            start_gather(j + K, (j + K) % NBUF)
            wait_gather(j, j % NBUF)
            start_out(j, j % NBUF)

        def body(gg, carry):
            for p in range(NBUF):
                j = NBUF * gg + K + p
                bg = (K + p + K) % NBUF
                bj = (K + p) % NBUF
                wait_out(j + K - NBUF, bg)
                start_gather(j + K, bg)
                wait_gather(j, bj)
                start_out(j, bj)
            return carry

        lax.fori_loop(0, (n_ch - 2 * K) // NBUF, body, 0)

        for j in range(n_ch - K, n_ch):
            wait_out(j + K - NBUF, (j + K) % NBUF)
            wait_gather(j, j % NBUF)
            start_out(j, j % NBUF)
        for j in range(n_ch - NBUF + K, n_ch):
            wait_out(j, j % NBUF)

    return k(table, idx3)


def kernel(pos_seq, table):
    B4, S = pos_seq.shape
    B = B4 * S
    NW = 32
    idx3 = pos_seq.astype(jnp.int32).reshape(NW, (B // NW) // CHUNK, CHUNK)
    out = _gather_sc(table, idx3)
    return out.reshape(B4, S, D_MODEL)
